# baseline (speedup 1.0000x reference)
import functools

import jax
import jax.numpy as jnp
from jax import lax
from jax.experimental import pallas as pl
from jax.experimental.pallas import tpu as pltpu
from jax.experimental.pallas import tpu_sc as plsc

_NC = 2
_NS = 16
_NW = _NC * _NS
_LANES = 16


def _make_embed(n_rows, n_hist, dim):
    bn = n_rows // _NW
    mesh = plsc.VectorSubcoreMesh(core_axis_name="c", subcore_axis_name="s")

    @functools.partial(
        pl.kernel,
        out_type=jax.ShapeDtypeStruct((n_hist, dim, n_rows), jnp.float32),
        mesh=mesh,
        scratch_types=[
            pltpu.VMEM((n_hist, bn), jnp.int32),
            pltpu.VMEM((bn, dim), jnp.float32),
            pltpu.VMEM((bn, dim), jnp.float32),
            pltpu.VMEM((dim, bn), jnp.float32),
            pltpu.VMEM((dim, bn), jnp.float32),
            pltpu.SemaphoreType.DMA,
            pltpu.SemaphoreType.DMA,
            pltpu.SemaphoreType.DMA,
            pltpu.SemaphoreType.DMA,
        ],
        compiler_params=pltpu.CompilerParams(
            use_tc_tiling_on_sc=False, needs_layout_passes=False),
    )
    def embed_kernel(xt_hbm, table_hbm, out_hbm, idx_t, rows0, rows1,
                     tb0, tb1, g0, g1, s0, s1):
        wid = lax.axis_index("s") * _NC + lax.axis_index("c")
        b0 = wid * bn

        rows = (rows0, rows1)
        tb = (tb0, tb1)
        gsem = (g0, g1)
        ssem = (s0, s1)

        pltpu.sync_copy(xt_hbm.at[:, pl.ds(b0, bn)], idx_t)

        def gather_desc(h, p):
            return pltpu.make_async_copy(
                table_hbm.at[idx_t.at[h, :]], rows[p], gsem[p])

        def store_desc(h, p):
            return pltpu.make_async_copy(
                tb[p], out_hbm.at[h, :, pl.ds(b0, bn)], ssem[p])

        iota = lax.iota(jnp.int32, _LANES)

        def transpose(p):
            def body(bg, carry):
                ridx = bg * _LANES + iota
                for d in range(dim):
                    v = plsc.load_gather(
                        rows[p], [ridx, jnp.full((_LANES,), d, jnp.int32)])
                    tb[p][d, pl.ds(bg * _LANES, _LANES)] = v
                return carry
            lax.fori_loop(0, bn // _LANES, body, 0)

        def step(h, p, next_gather, wait_store):
            gather_desc(h, p).wait()
            if next_gather:
                gather_desc(h + 1, 1 - p).start()
            if wait_store:
                store_desc(h, p).wait()
            transpose(p)
            store_desc(h, p).start()

        gather_desc(0, 0).start()
        step(0, 0, True, False)
        step(1, 1, True, False)

        def steady(i, carry):
            h = 2 + 2 * i
            step(h, 0, True, True)
            step(h + 1, 1, True, True)
            return carry

        lax.fori_loop(0, (n_hist - 4) // 2, steady, 0)
        step(n_hist - 2, 0, True, True)
        step(n_hist - 1, 1, False, True)
        store_desc(n_hist - 2, 0).wait()
        store_desc(n_hist - 1, 1).wait()

    return embed_kernel


def kernel(x, table):
    xt = x.T
    out_t = _make_embed(x.shape[0], x.shape[1], table.shape[1])(xt, table)
    return out_t.transpose(2, 0, 1)


# speedup vs baseline: 1.2707x; 1.2707x over previous
import functools

import jax
import jax.numpy as jnp
from jax import lax
from jax.experimental import pallas as pl
from jax.experimental.pallas import tpu as pltpu
from jax.experimental.pallas import tpu_sc as plsc

_NC = 2
_NS = 16
_NW = _NC * _NS
_LANES = 16


def _make_embed(n_rows, n_hist, dim):
    bn = n_rows // _NW
    mesh = plsc.VectorSubcoreMesh(core_axis_name="c", subcore_axis_name="s")

    @functools.partial(
        pl.kernel,
        out_type=jax.ShapeDtypeStruct((n_hist, dim, n_rows), jnp.float32),
        mesh=mesh,
        scratch_types=[
            pltpu.VMEM((n_hist, bn), jnp.int32),
            pltpu.VMEM((bn, dim), jnp.float32),
            pltpu.VMEM((bn, dim), jnp.float32),
            pltpu.VMEM((dim, bn), jnp.float32),
            pltpu.VMEM((dim, bn), jnp.float32),
            pltpu.SemaphoreType.DMA,
            pltpu.SemaphoreType.DMA,
            pltpu.SemaphoreType.DMA,
            pltpu.SemaphoreType.DMA,
        ],
        compiler_params=pltpu.CompilerParams(
            use_tc_tiling_on_sc=False, needs_layout_passes=False),
    )
    def embed_kernel(xt_hbm, table_hbm, out_hbm, idx_t, rows0, rows1,
                     tb0, tb1, g0, g1, s0, s1):
        wid = lax.axis_index("s") * _NC + lax.axis_index("c")
        b0 = wid * bn

        rows = (rows0, rows1)
        tb = (tb0, tb1)
        gsem = (g0, g1)
        ssem = (s0, s1)

        pltpu.sync_copy(xt_hbm.at[:, pl.ds(b0, bn)], idx_t)

        def gather_desc(h, p):
            return pltpu.make_async_copy(
                table_hbm.at[idx_t.at[h, :]], rows[p], gsem[p])

        def store_desc(h, p):
            return pltpu.make_async_copy(
                tb[p], out_hbm.at[h, :, pl.ds(b0, bn)], ssem[p])

        iota = lax.iota(jnp.int32, _LANES)

        def transpose(p):
            def body(bg, carry):
                ridx = bg * _LANES + iota
                vs = [plsc.load_gather(
                    rows[p], [ridx, jnp.full((_LANES,), d, jnp.int32)])
                    for d in range(dim)]
                for d in range(dim):
                    tb[p][d, pl.ds(bg * _LANES, _LANES)] = vs[d]
                return carry
            lax.fori_loop(0, bn // _LANES, body, 0)

        def step(h, p, next_gather, wait_store):
            gather_desc(h, p).wait()
            if next_gather:
                gather_desc(h + 1, 1 - p).start()
            if wait_store:
                store_desc(h, p).wait()
            transpose(p)
            store_desc(h, p).start()

        gather_desc(0, 0).start()
        step(0, 0, True, False)
        step(1, 1, True, False)

        def steady(i, carry):
            h = 2 + 2 * i
            step(h, 0, True, True)
            step(h + 1, 1, True, True)
            return carry

        lax.fori_loop(0, (n_hist - 4) // 2, steady, 0)
        step(n_hist - 2, 0, True, True)
        step(n_hist - 1, 1, False, True)
        store_desc(n_hist - 2, 0).wait()
        store_desc(n_hist - 1, 1).wait()

    return embed_kernel


def kernel(x, table):
    xt = x.T
    out_t = _make_embed(x.shape[0], x.shape[1], table.shape[1])(xt, table)
    return out_t.transpose(2, 0, 1)


# speedup vs baseline: 1.2934x; 1.0179x over previous
import functools

import jax
import jax.numpy as jnp
from jax import lax
from jax.experimental import pallas as pl
from jax.experimental.pallas import tpu as pltpu
from jax.experimental.pallas import tpu_sc as plsc

_NC = 2
_NS = 16
_NW = _NC * _NS
_LANES = 16


def _make_embed(n_rows, n_hist, dim):
    bn = n_rows // _NW
    mesh = plsc.VectorSubcoreMesh(core_axis_name="c", subcore_axis_name="s")

    @functools.partial(
        pl.kernel,
        out_type=jax.ShapeDtypeStruct((n_hist, dim, n_rows), jnp.float32),
        mesh=mesh,
        scratch_types=[
            pltpu.VMEM((n_hist, bn), jnp.int32),
            pltpu.VMEM((bn, dim), jnp.float32),
            pltpu.VMEM((bn, dim), jnp.float32),
            pltpu.VMEM((dim, bn), jnp.float32),
            pltpu.VMEM((dim, bn), jnp.float32),
            pltpu.SemaphoreType.DMA,
            pltpu.SemaphoreType.DMA,
            pltpu.SemaphoreType.DMA,
            pltpu.SemaphoreType.DMA,
        ],
        compiler_params=pltpu.CompilerParams(
            use_tc_tiling_on_sc=False, needs_layout_passes=False),
    )
    def embed_kernel(xt_hbm, table_hbm, out_hbm, idx_t, rows0, rows1,
                     tb0, tb1, g0, g1, s0, s1):
        wid = lax.axis_index("s") * _NC + lax.axis_index("c")
        b0 = wid * bn

        rows = (rows0, rows1)
        tb = (tb0, tb1)
        gsem = (g0, g1)
        ssem = (s0, s1)

        pltpu.sync_copy(xt_hbm.at[:, pl.ds(b0, bn)], idx_t)

        def gather_desc(h, p):
            return pltpu.make_async_copy(
                table_hbm.at[idx_t.at[h, :]], rows[p], gsem[p])

        def store_desc(h, p):
            return pltpu.make_async_copy(
                tb[p], out_hbm.at[h, :, pl.ds(b0, bn)], ssem[p])

        iota = lax.iota(jnp.int32, _LANES)

        def transpose(p):
            @plsc.parallel_loop(0, bn // _LANES, unroll=2)
            def body(bg):
                ridx = bg * _LANES + iota
                vs = [plsc.load_gather(
                    rows[p], [ridx, jnp.full((_LANES,), d, jnp.int32)])
                    for d in range(dim)]
                for d in range(dim):
                    tb[p][d, pl.ds(bg * _LANES, _LANES)] = vs[d]

        def step(h, p, next_gather, wait_store):
            gather_desc(h, p).wait()
            if next_gather:
                gather_desc(h + 1, 1 - p).start()
            if wait_store:
                store_desc(h, p).wait()
            transpose(p)
            store_desc(h, p).start()

        gather_desc(0, 0).start()
        step(0, 0, True, False)
        step(1, 1, True, False)

        def steady(i, carry):
            h = 2 + 2 * i
            step(h, 0, True, True)
            step(h + 1, 1, True, True)
            return carry

        lax.fori_loop(0, (n_hist - 4) // 2, steady, 0)
        step(n_hist - 2, 0, True, True)
        step(n_hist - 1, 1, False, True)
        store_desc(n_hist - 2, 0).wait()
        store_desc(n_hist - 1, 1).wait()

    return embed_kernel


def kernel(x, table):
    xt = x.T
    out_t = _make_embed(x.shape[0], x.shape[1], table.shape[1])(xt, table)
    return out_t.transpose(2, 0, 1)
